# baseline (speedup 1.0000x reference)
import numpy as np
import jax
import jax.numpy as jnp
from jax.experimental import pallas as pl
from jax.experimental.pallas import tpu as pltpu

_H = _W = 64
_NA = 9
_A = _NA * _H * _W
_ROWS = _A // 128
_NUM_FG = 256 // 3
_NUM_BG = 256 * 2 // 3
_POS = 0.7
_NEG = 0.3
_G = 50


def _anchor_consts():
    sizes = [4.0, 8.0, 16.0]
    ratios = [0.5, 1.0, 2.0]
    ws, hs = [], []
    for s in sizes:
        for r in ratios:
            ws.append(s * np.sqrt(r))
            hs.append(s / np.sqrt(r))
    ws = np.asarray(ws, np.float32)
    hs = np.asarray(hs, np.float32)
    yy, xx = np.meshgrid(np.arange(_H, dtype=np.float32),
                         np.arange(_W, dtype=np.float32), indexing='ij')
    x = xx[None] + 0.5 - ws[:, None, None] / 2.0
    y = yy[None] + 0.5 - hs[:, None, None] / 2.0
    w = np.broadcast_to(ws[:, None, None], (_NA, _H, _W)).astype(np.float32)
    h = np.broadcast_to(hs[:, None, None], (_NA, _H, _W)).astype(np.float32)

    x2k = x + w - 1.0
    y2k = y + h - 1.0
    L = float(_H) - 1.0
    keep = (x >= 0) & (y >= 0) & (x2k >= 0) & (y2k >= 0)
    keep &= (x <= L) & (y <= L) & (x2k <= L) & (y2k <= L)
    keep &= (w >= 0) & (h >= 0) & (w <= L) & (h <= L)

    rs = lambda a: a.reshape(_ROWS, 128).astype(np.float32)
    return (rs(x), rs(y), rs(x + w), rs(y + h), rs(w * h),
            rs(keep.astype(np.float32)))


def _tri_consts():
    u = np.tril(np.ones((128, 128), np.float32)).T
    t = np.tril(np.ones((_ROWS, _ROWS), np.float32), k=-1)
    return u, t


_CH = 32
_NCH = _ROWS // _CH


def _body(gt_ref, x1_ref, y1_ref, x2_ref, y2_ref, area_ref, keep_ref,
          u_ref, t_ref, lab_ref, tgt_ref, mov_ref):
    def chunk_step(c, _):
        sl = pl.ds(c * _CH, _CH)
        ax1 = x1_ref[sl, :]
        ay1 = y1_ref[sl, :]
        ax2 = x2_ref[sl, :]
        ay2 = y2_ref[sl, :]
        area_a = area_ref[sl, :]

        def g_step(g, mx):
            gx1 = gt_ref[0, 0, g]
            gy1 = gt_ref[0, 1, g]
            gx2 = gt_ref[0, 2, g]
            gy2 = gt_ref[0, 3, g]
            ga = gt_ref[0, 4, g]
            ix = jnp.maximum(jnp.minimum(ax2, gx2) - jnp.maximum(ax1, gx1),
                             0.0)
            iy = jnp.maximum(jnp.minimum(ay2, gy2) - jnp.maximum(ay1, gy1),
                             0.0)
            inter = ix * iy
            iou = inter / jnp.maximum(area_a + ga - inter, 1e-10)
            return jnp.maximum(mx, iou)

        mov_ref[sl, :] = jax.lax.fori_loop(
            0, _G, g_step, jnp.zeros((_CH, 128), jnp.float32), unroll=2)
        return 0

    jax.lax.fori_loop(0, _NCH, chunk_step, 0)
    max_ov = mov_ref[...]

    kb = keep_ref[...] > 0.0
    is_fg = kb & (max_ov >= _POS)
    is_bg = kb & (max_ov <= _NEG)

    def rank_of(m):
        f = m.astype(jnp.float32)
        incl = jnp.dot(f, u_ref[...], preferred_element_type=jnp.float32)
        rowtot = jax.lax.broadcast_in_dim(incl[:, 127:128], (_ROWS, 128),
                                          (0, 1))
        pref = jnp.dot(t_ref[...], rowtot,
                       preferred_element_type=jnp.float32)
        return incl + pref

    fg_ok = is_fg & (rank_of(is_fg) <= float(_NUM_FG))
    bg_ok = is_bg & (rank_of(is_bg) <= float(_NUM_BG))
    lab = jnp.where(fg_ok, 1.0, jnp.where(bg_ok, 0.0, -1.0))
    lab_ref[0] = lab
    tgt_ref[0] = jnp.zeros((_ROWS, 512), jnp.float32)


def kernel(cls_scores, gt_boxes, image_info):
    B = gt_boxes.shape[0]
    x1, y1, x2, y2, area, keep = (jnp.asarray(a) for a in _anchor_consts())
    u128, t288 = (jnp.asarray(a) for a in _tri_consts())

    gx1 = gt_boxes[:, :, 0]
    gy1 = gt_boxes[:, :, 1]
    gx2 = gx1 + gt_boxes[:, :, 2]
    gy2 = gy1 + gt_boxes[:, :, 3]
    ga = gt_boxes[:, :, 2] * gt_boxes[:, :, 3]
    gt = jnp.stack([gx1, gy1, gx2, gy2, ga], axis=1)
    gt = jnp.pad(gt, ((0, 0), (0, 3), (0, 64 - _G)))

    full = lambda shp: pl.BlockSpec(shp, lambda b: (0,) * len(shp))
    lab, tgt = pl.pallas_call(
        _body,
        grid=(B,),
        in_specs=[
            pl.BlockSpec((1, 8, 64), lambda b: (b, 0, 0),
                         memory_space=pltpu.SMEM),
            full((_ROWS, 128)), full((_ROWS, 128)), full((_ROWS, 128)),
            full((_ROWS, 128)), full((_ROWS, 128)), full((_ROWS, 128)),
            full((128, 128)), full((_ROWS, _ROWS)),
        ],
        out_specs=[
            pl.BlockSpec((1, _ROWS, 128), lambda b: (b, 0, 0)),
            pl.BlockSpec((1, _ROWS, 512), lambda b: (b, 0, 0)),
        ],
        out_shape=[
            jax.ShapeDtypeStruct((B, _ROWS, 128), jnp.float32),
            jax.ShapeDtypeStruct((B, _ROWS, 512), jnp.float32),
        ],
        scratch_shapes=[pltpu.VMEM((_ROWS, 128), jnp.float32)],
        compiler_params=pltpu.CompilerParams(
            dimension_semantics=("parallel",)),
    )(gt, x1, y1, x2, y2, area, keep, u128, t288)

    return (lab.reshape(B, _NA, _H, _W, 1), tgt.reshape(B, _NA, _H, _W, 4))


# speedup vs baseline: 1.2002x; 1.2002x over previous
import numpy as np
import jax
import jax.numpy as jnp
from jax.experimental import pallas as pl
from jax.experimental.pallas import tpu as pltpu

_H = _W = 64
_NA = 9
_A = _NA * _H * _W
_ROWS = _A // 128
_NUM_FG = 256 // 3
_NUM_BG = 256 * 2 // 3
_POS = 0.7
_NEG = 0.3
_G = 50


def _anchor_consts():
    sizes = [4.0, 8.0, 16.0]
    ratios = [0.5, 1.0, 2.0]
    ws, hs = [], []
    for s in sizes:
        for r in ratios:
            ws.append(s * np.sqrt(r))
            hs.append(s / np.sqrt(r))
    ws = np.asarray(ws, np.float32)
    hs = np.asarray(hs, np.float32)
    yy, xx = np.meshgrid(np.arange(_H, dtype=np.float32),
                         np.arange(_W, dtype=np.float32), indexing='ij')
    x = xx[None] + 0.5 - ws[:, None, None] / 2.0
    y = yy[None] + 0.5 - hs[:, None, None] / 2.0
    w = np.broadcast_to(ws[:, None, None], (_NA, _H, _W)).astype(np.float32)
    h = np.broadcast_to(hs[:, None, None], (_NA, _H, _W)).astype(np.float32)

    x2k = x + w - 1.0
    y2k = y + h - 1.0
    L = float(_H) - 1.0
    keep = (x >= 0) & (y >= 0) & (x2k >= 0) & (y2k >= 0)
    keep &= (x <= L) & (y <= L) & (x2k <= L) & (y2k <= L)
    keep &= (w >= 0) & (h >= 0) & (w <= L) & (h <= L)

    rs = lambda a: a.reshape(_ROWS, 128).astype(np.float32)
    return (rs(x), rs(y), rs(x + w), rs(y + h), rs(w * h),
            rs(keep.astype(np.float32)))


def _tri_consts():
    u = np.tril(np.ones((128, 128), np.float32)).T
    t = np.tril(np.ones((_ROWS, _ROWS), np.float32), k=-1)
    return u, t


_CH = 32
_NCH = _ROWS // _CH


def _body(gt_ref, x1_ref, y1_ref, x2_ref, y2_ref, area_ref, keep_ref,
          u_ref, t_ref, lab_ref, tgt_ref, mov_ref):
    def chunk_step(c, _):
        sl = pl.ds(c * _CH, _CH)
        ax1 = x1_ref[sl, :]
        ay1 = y1_ref[sl, :]
        ax2 = x2_ref[sl, :]
        ay2 = y2_ref[sl, :]
        area_a = area_ref[sl, :]

        def g_step(g, mx):
            gx1 = gt_ref[0, 0, g]
            gy1 = gt_ref[0, 1, g]
            gx2 = gt_ref[0, 2, g]
            gy2 = gt_ref[0, 3, g]
            ga = gt_ref[0, 4, g]
            ix = jnp.maximum(jnp.minimum(ax2, gx2) - jnp.maximum(ax1, gx1),
                             0.0)
            iy = jnp.maximum(jnp.minimum(ay2, gy2) - jnp.maximum(ay1, gy1),
                             0.0)
            inter = ix * iy
            iou = inter / jnp.maximum(area_a + ga - inter, 1e-10)
            return jnp.maximum(mx, iou)

        mov_ref[sl, :] = jax.lax.fori_loop(
            0, _G, g_step, jnp.zeros((_CH, 128), jnp.float32), unroll=5)
        return 0

    jax.lax.fori_loop(0, _NCH, chunk_step, 0)
    max_ov = mov_ref[...]

    kb = keep_ref[...] > 0.0
    is_fg = kb & (max_ov >= _POS)
    is_bg = kb & (max_ov <= _NEG)

    def rank_of(m):
        f = m.astype(jnp.float32)
        incl = jnp.dot(f, u_ref[...], preferred_element_type=jnp.float32)
        rowtot = jax.lax.broadcast_in_dim(incl[:, 127:128], (_ROWS, 128),
                                          (0, 1))
        pref = jnp.dot(t_ref[...], rowtot,
                       preferred_element_type=jnp.float32)
        return incl + pref

    fg_ok = is_fg & (rank_of(is_fg) <= float(_NUM_FG))
    bg_ok = is_bg & (rank_of(is_bg) <= float(_NUM_BG))
    lab = jnp.where(fg_ok, 1.0, jnp.where(bg_ok, 0.0, -1.0))
    lab_ref[0] = lab
    tgt_ref[0] = jnp.zeros((_ROWS, 512), jnp.float32)


def kernel(cls_scores, gt_boxes, image_info):
    B = gt_boxes.shape[0]
    x1, y1, x2, y2, area, keep = (jnp.asarray(a) for a in _anchor_consts())
    u128, t288 = (jnp.asarray(a) for a in _tri_consts())

    gx1 = gt_boxes[:, :, 0]
    gy1 = gt_boxes[:, :, 1]
    gx2 = gx1 + gt_boxes[:, :, 2]
    gy2 = gy1 + gt_boxes[:, :, 3]
    ga = gt_boxes[:, :, 2] * gt_boxes[:, :, 3]
    gt = jnp.stack([gx1, gy1, gx2, gy2, ga], axis=1)
    gt = jnp.pad(gt, ((0, 0), (0, 3), (0, 64 - _G)))

    full = lambda shp: pl.BlockSpec(shp, lambda b: (0,) * len(shp))
    lab, tgt = pl.pallas_call(
        _body,
        grid=(B,),
        in_specs=[
            pl.BlockSpec((1, 8, 64), lambda b: (b, 0, 0),
                         memory_space=pltpu.SMEM),
            full((_ROWS, 128)), full((_ROWS, 128)), full((_ROWS, 128)),
            full((_ROWS, 128)), full((_ROWS, 128)), full((_ROWS, 128)),
            full((128, 128)), full((_ROWS, _ROWS)),
        ],
        out_specs=[
            pl.BlockSpec((1, _ROWS, 128), lambda b: (b, 0, 0)),
            pl.BlockSpec((1, _ROWS, 512), lambda b: (b, 0, 0)),
        ],
        out_shape=[
            jax.ShapeDtypeStruct((B, _ROWS, 128), jnp.float32),
            jax.ShapeDtypeStruct((B, _ROWS, 512), jnp.float32),
        ],
        scratch_shapes=[pltpu.VMEM((_ROWS, 128), jnp.float32)],
        compiler_params=pltpu.CompilerParams(
            dimension_semantics=("parallel",)),
    )(gt, x1, y1, x2, y2, area, keep, u128, t288)

    return (lab.reshape(B, _NA, _H, _W, 1), tgt.reshape(B, _NA, _H, _W, 4))


# speedup vs baseline: 1.8525x; 1.5434x over previous
import functools

import numpy as np
import jax
import jax.numpy as jnp
from jax import lax
from jax.experimental import pallas as pl
from jax.experimental.pallas import tpu as pltpu
from jax.experimental.pallas import tpu_sc as plsc

_H = _W = 64
_NA = 9
_A = _NA * _H * _W
_ROWS = _A // 128
_NUM_FG = 256 // 3
_NUM_BG = 256 * 2 // 3
_POS = 0.7
_NEG = 0.3
_G = 50
_RATIO = float(0.7 * 1.3 / (0.3 * 1.7))


def _anchor_consts():
    sizes = [4.0, 8.0, 16.0]
    ratios = [0.5, 1.0, 2.0]
    ws, hs = [], []
    for s in sizes:
        for r in ratios:
            ws.append(s * np.sqrt(r))
            hs.append(s / np.sqrt(r))
    ws = np.asarray(ws, np.float32)
    hs = np.asarray(hs, np.float32)
    yy, xx = np.meshgrid(np.arange(_H, dtype=np.float32),
                         np.arange(_W, dtype=np.float32), indexing='ij')
    x = xx[None] + 0.5 - ws[:, None, None] / 2.0
    y = yy[None] + 0.5 - hs[:, None, None] / 2.0
    w = np.broadcast_to(ws[:, None, None], (_NA, _H, _W)).astype(np.float32)
    h = np.broadcast_to(hs[:, None, None], (_NA, _H, _W)).astype(np.float32)

    x2k = x + w - 1.0
    y2k = y + h - 1.0
    L = float(_H) - 1.0
    keep = (x >= 0) & (y >= 0) & (x2k >= 0) & (y2k >= 0)
    keep &= (x <= L) & (y <= L) & (x2k <= L) & (y2k <= L)
    keep &= (w >= 0) & (h >= 0) & (w <= L) & (h <= L)

    rs = lambda a: a.reshape(_ROWS, 128).astype(np.float32)
    xs = np.arange(_W, dtype=np.float32)
    x1b = (xs[None] + 0.5 - ws[:, None] / 2.0).astype(np.float32)
    x2b = (x1b + ws[:, None]).astype(np.float32)
    pad = np.zeros((16 - _NA, 128), np.float32)
    x1r = np.concatenate([np.tile(x1b, (1, 2)), pad], 0)
    x2r = np.concatenate([np.tile(x2b, (1, 2)), pad], 0)
    areas = (ws * hs).astype(np.float32)
    return (rs(y), rs(y + h), rs(keep.astype(np.float32)), x1r, x2r, areas)


def _tri_consts():
    u = np.tril(np.ones((128, 128), np.bfloat16 if False else np.float32)).T
    t = np.tril(np.ones((_ROWS, _ROWS), np.float32), k=-1)
    return u.astype('bfloat16'), t.astype('bfloat16')


_CH = 32
_NCH = _ROWS // _CH


def _body(gt_ref, inv_ref, y1_ref, y2_ref, keep_ref, x1r_ref, x2r_ref,
          u_ref, t_ref, lab_ref, mov_ref):
    def chunk_step(c, _):
        sl = pl.ds(c * _CH, _CH)
        ay1 = y1_ref[sl, :]
        ay2 = y2_ref[sl, :]
        ax1 = x1r_ref[pl.ds(c, 1), :]
        ax2 = x2r_ref[pl.ds(c, 1), :]

        def g_step(g, mx):
            gx1 = gt_ref[0, 0, g]
            gy1 = gt_ref[0, 1, g]
            gx2 = gt_ref[0, 2, g]
            gy2 = gt_ref[0, 3, g]
            s = inv_ref[0, c, g]
            ix = jnp.maximum(jnp.minimum(ax2, gx2) - jnp.maximum(ax1, gx1),
                             0.0)
            iy = jnp.maximum(jnp.minimum(ay2, gy2) - jnp.maximum(ay1, gy1),
                             0.0)
            return jnp.maximum(mx, (iy * ix) * s)

        mov_ref[sl, :] = jax.lax.fori_loop(
            0, _G, g_step, jnp.zeros((_CH, 128), jnp.float32), unroll=5)
        return 0

    jax.lax.fori_loop(0, _NCH, chunk_step, 0)
    m = mov_ref[...]

    kb = keep_ref[...] > 0.0
    lab_ref[0] = jnp.where(kb, m, 1.5)


def kernel(cls_scores, gt_boxes, image_info):
    B = gt_boxes.shape[0]
    y1c, y2c, keep, x1r, x2r, areas = _anchor_consts()
    y1c, y2c, keep, x1r, x2r = (jnp.asarray(a)
                                for a in (y1c, y2c, keep, x1r, x2r))
    u128, t288 = (jnp.asarray(a) for a in _tri_consts())

    gx1 = gt_boxes[:, :, 0]
    gy1 = gt_boxes[:, :, 1]
    gx2 = gx1 + gt_boxes[:, :, 2]
    gy2 = gy1 + gt_boxes[:, :, 3]
    gt = jnp.stack([gx1, gy1, gx2, gy2], axis=1)
    gt = jnp.pad(gt, ((0, 0), (0, 4), (0, 64 - _G)))

    ga = gt_boxes[:, :, 2] * gt_boxes[:, :, 3]
    k2 = np.float32(0.3 / 1.3)
    inv = 1.0 / (k2 * (jnp.asarray(areas)[None, :, None]
                       + ga[:, None, :]))
    inv = jnp.pad(inv.astype(jnp.float32),
                  ((0, 0), (0, 16 - _NA), (0, 64 - _G)))

    full = lambda shp: pl.BlockSpec(shp, lambda b: (0,) * len(shp))
    (lab,) = pl.pallas_call(
        _body,
        grid=(B,),
        in_specs=[
            pl.BlockSpec((1, 8, 64), lambda b: (b, 0, 0),
                         memory_space=pltpu.SMEM),
            pl.BlockSpec((1, 16, 64), lambda b: (b, 0, 0),
                         memory_space=pltpu.SMEM),
            full((_ROWS, 128)), full((_ROWS, 128)), full((_ROWS, 128)),
            full((16, 128)), full((16, 128)),
            full((128, 128)), full((_ROWS, _ROWS)),
        ],
        out_specs=[
            pl.BlockSpec((1, _ROWS, 128), lambda b: (b, 0, 0)),
        ],
        out_shape=[
            jax.ShapeDtypeStruct((B, _ROWS, 128), jnp.float32),
        ],
        scratch_shapes=[pltpu.VMEM((_ROWS, 128), jnp.float32)],
        compiler_params=pltpu.CompilerParams(
            dimension_semantics=("parallel",)),
    )(gt, inv, y1c, y2c, keep, x1r, x2r, u128, t288)

    lab = _sc_balance(lab.reshape(B * _A))
    tgt = jnp.zeros((B, _NA, _H, _W, 4), jnp.float32)
    return (lab.reshape(B, _NA, _H, _W, 1), tgt)


def _sc_balance(mflat):
    B = mflat.shape[0] // _A

    @functools.partial(
        pl.kernel,
        out_type=jax.ShapeDtypeStruct((B * _A,), jnp.float32),
        mesh=plsc.VectorSubcoreMesh(core_axis_name="c", subcore_axis_name="s"),
        scratch_types=[pltpu.VMEM((_A,), jnp.float32),
                       pltpu.SemaphoreType.DMA],
        compiler_params=pltpu.CompilerParams(needs_layout_passes=False),
    )
    def bal(m_hbm, out_hbm, buf, sem):
        wid = lax.axis_index("s") * 2 + lax.axis_index("c")

        @pl.when(wid < B)
        def _():
            base = wid * _A
            pltpu.async_copy(m_hbm.at[pl.ds(base, _A)], buf, sem).wait()

            def step(i, carry):
                rf, rb = carry
                v = buf[pl.ds(i * 16, 16)]
                fg = v >= _RATIO
                bg = v <= 1.0
                fgf = jnp.where(fg, 1.0, 0.0)
                bgf = jnp.where(bg, 1.0, 0.0)
                cf = plsc.cumsum(fgf) + rf
                cb = plsc.cumsum(bgf) + rb
                lab = jnp.where(fg & (cf <= float(_NUM_FG)), 1.0,
                                jnp.where(bg & (cb <= float(_NUM_BG)),
                                          0.0, -1.0))
                buf[pl.ds(i * 16, 16)] = lab
                return (rf + jnp.sum(fgf), rb + jnp.sum(bgf))

            lax.fori_loop(0, _A // 16, step, (0.0, 0.0), unroll=4)
            pltpu.async_copy(buf, out_hbm.at[pl.ds(base, _A)], sem).wait()

    return bal(mflat)


# speedup vs baseline: 1.8549x; 1.0013x over previous
import functools

import numpy as np
import jax
import jax.numpy as jnp
from jax import lax
from jax.experimental import pallas as pl
from jax.experimental.pallas import tpu as pltpu
from jax.experimental.pallas import tpu_sc as plsc

_H = _W = 64
_NA = 9
_A = _NA * _H * _W
_ROWS = _A // 128
_NUM_FG = 256 // 3
_NUM_BG = 256 * 2 // 3
_POS = 0.7
_NEG = 0.3
_G = 50
_RATIO = float(0.7 * 1.3 / (0.3 * 1.7))


def _anchor_consts():
    sizes = [4.0, 8.0, 16.0]
    ratios = [0.5, 1.0, 2.0]
    ws, hs = [], []
    for s in sizes:
        for r in ratios:
            ws.append(s * np.sqrt(r))
            hs.append(s / np.sqrt(r))
    ws = np.asarray(ws, np.float32)
    hs = np.asarray(hs, np.float32)
    yy, xx = np.meshgrid(np.arange(_H, dtype=np.float32),
                         np.arange(_W, dtype=np.float32), indexing='ij')
    x = xx[None] + 0.5 - ws[:, None, None] / 2.0
    y = yy[None] + 0.5 - hs[:, None, None] / 2.0
    w = np.broadcast_to(ws[:, None, None], (_NA, _H, _W)).astype(np.float32)
    h = np.broadcast_to(hs[:, None, None], (_NA, _H, _W)).astype(np.float32)

    x2k = x + w - 1.0
    y2k = y + h - 1.0
    L = float(_H) - 1.0
    keep = (x >= 0) & (y >= 0) & (x2k >= 0) & (y2k >= 0)
    keep &= (x <= L) & (y <= L) & (x2k <= L) & (y2k <= L)
    keep &= (w >= 0) & (h >= 0) & (w <= L) & (h <= L)

    rs = lambda a: a.reshape(_ROWS, 128).astype(np.float32)
    xs = np.arange(_W, dtype=np.float32)
    x1b = (xs[None] + 0.5 - ws[:, None] / 2.0).astype(np.float32)
    x2b = (x1b + ws[:, None]).astype(np.float32)
    pad = np.zeros((16 - _NA, 128), np.float32)
    x1r = np.concatenate([np.tile(x1b, (1, 2)), pad], 0)
    x2r = np.concatenate([np.tile(x2b, (1, 2)), pad], 0)
    areas = (ws * hs).astype(np.float32)
    return (rs(y), rs(y + h), rs(keep.astype(np.float32)), x1r, x2r, areas)


_CH = 32
_NCH = _ROWS // _CH


def _body(gt_ref, inv_ref, y1_ref, y2_ref, keep_ref, x1r_ref, x2r_ref,
          lab_ref, mov_ref):
    def chunk_step(c, _):
        sl = pl.ds(c * _CH, _CH)
        ay1 = y1_ref[sl, :]
        ay2 = y2_ref[sl, :]
        ax1 = x1r_ref[pl.ds(c, 1), :]
        ax2 = x2r_ref[pl.ds(c, 1), :]

        def g_step(g, mx):
            gx1 = gt_ref[0, 0, g]
            gy1 = gt_ref[0, 1, g]
            gx2 = gt_ref[0, 2, g]
            gy2 = gt_ref[0, 3, g]
            s = inv_ref[0, c, g]
            ix = jnp.maximum(jnp.minimum(ax2, gx2) - jnp.maximum(ax1, gx1),
                             0.0)
            iy = jnp.maximum(jnp.minimum(ay2, gy2) - jnp.maximum(ay1, gy1),
                             0.0)
            return jnp.maximum(mx, (iy * ix) * s)

        mov_ref[sl, :] = jax.lax.fori_loop(
            0, _G, g_step, jnp.zeros((_CH, 128), jnp.float32), unroll=5)
        return 0

    jax.lax.fori_loop(0, _NCH, chunk_step, 0)
    m = mov_ref[...]

    kb = keep_ref[...] > 0.0
    lab_ref[0] = jnp.where(kb, m, 1.5)


def kernel(cls_scores, gt_boxes, image_info):
    B = gt_boxes.shape[0]
    y1c, y2c, keep, x1r, x2r, areas = _anchor_consts()
    y1c, y2c, keep, x1r, x2r = (jnp.asarray(a)
                                for a in (y1c, y2c, keep, x1r, x2r))

    gx1 = gt_boxes[:, :, 0]
    gy1 = gt_boxes[:, :, 1]
    gx2 = gx1 + gt_boxes[:, :, 2]
    gy2 = gy1 + gt_boxes[:, :, 3]
    gt = jnp.stack([gx1, gy1, gx2, gy2], axis=1)
    gt = jnp.pad(gt, ((0, 0), (0, 4), (0, 64 - _G)))

    ga = gt_boxes[:, :, 2] * gt_boxes[:, :, 3]
    k2 = np.float32(0.3 / 1.3)
    inv = 1.0 / (k2 * (jnp.asarray(areas)[None, :, None]
                       + ga[:, None, :]))
    inv = jnp.pad(inv.astype(jnp.float32),
                  ((0, 0), (0, 16 - _NA), (0, 64 - _G)))

    full = lambda shp: pl.BlockSpec(shp, lambda b: (0,) * len(shp))
    (lab,) = pl.pallas_call(
        _body,
        grid=(B,),
        in_specs=[
            pl.BlockSpec((1, 8, 64), lambda b: (b, 0, 0),
                         memory_space=pltpu.SMEM),
            pl.BlockSpec((1, 16, 64), lambda b: (b, 0, 0),
                         memory_space=pltpu.SMEM),
            full((_ROWS, 128)), full((_ROWS, 128)), full((_ROWS, 128)),
            full((16, 128)), full((16, 128)),
        ],
        out_specs=[
            pl.BlockSpec((1, _ROWS, 128), lambda b: (b, 0, 0)),
        ],
        out_shape=[
            jax.ShapeDtypeStruct((B, _ROWS, 128), jnp.float32),
        ],
        scratch_shapes=[pltpu.VMEM((_ROWS, 128), jnp.float32)],
        compiler_params=pltpu.CompilerParams(
            dimension_semantics=("parallel",)),
    )(gt, inv, y1c, y2c, keep, x1r, x2r)

    lab = _sc_balance(lab.reshape(B * _A))
    tgt = jnp.zeros((B, _NA, _H, _W, 4), jnp.float32)
    return (lab.reshape(B, _NA, _H, _W, 1), tgt)


def _sc_balance(mflat):
    B = mflat.shape[0] // _A

    @functools.partial(
        pl.kernel,
        out_type=jax.ShapeDtypeStruct((B * _A,), jnp.float32),
        mesh=plsc.VectorSubcoreMesh(core_axis_name="c", subcore_axis_name="s"),
        scratch_types=[pltpu.VMEM((_A,), jnp.float32),
                       pltpu.SemaphoreType.DMA],
        compiler_params=pltpu.CompilerParams(needs_layout_passes=False),
    )
    def bal(m_hbm, out_hbm, buf, sem):
        wid = lax.axis_index("s") * 2 + lax.axis_index("c")

        @pl.when(wid < B)
        def _():
            base = wid * _A
            pltpu.async_copy(m_hbm.at[pl.ds(base, _A)], buf, sem).wait()

            def step(i, carry):
                rf, rb = carry
                v = buf[pl.ds(i * 16, 16)]
                fg = v >= _RATIO
                bg = v <= 1.0
                fgf = jnp.where(fg, 1.0, 0.0)
                bgf = jnp.where(bg, 1.0, 0.0)
                cf = plsc.cumsum(fgf) + rf
                cb = plsc.cumsum(bgf) + rb
                lab = jnp.where(fg & (cf <= float(_NUM_FG)), 1.0,
                                jnp.where(bg & (cb <= float(_NUM_BG)),
                                          0.0, -1.0))
                buf[pl.ds(i * 16, 16)] = lab
                return (rf + jnp.sum(fgf), rb + jnp.sum(bgf))

            lax.fori_loop(0, _A // 16, step, (0.0, 0.0), unroll=4)
            pltpu.async_copy(buf, out_hbm.at[pl.ds(base, _A)], sem).wait()

    return bal(mflat)
